# Initial kernel scaffold; baseline (speedup 1.0000x reference)
import jax
import jax.numpy as jnp
from jax.experimental import pallas as pl


def kernel(x, edge_index, edge_type, W_rel, W_self):
    raise NotImplementedError("write your pallas kernel here")


# speedup vs baseline: 2.1177x; 2.1177x over previous
import functools

import jax
import jax.numpy as jnp
from jax import lax
from jax.experimental import pallas as pl
from jax.experimental.pallas import tpu as pltpu
from jax.experimental.pallas import tpu_sc as plsc

N = 10000
E = 320000
D = 128
R = 8

NC = 2
NS = 16
NW = NC * NS

CH = 128
CPT = 8 * -(-E // (CH * NW * 8))
NCHUNK = CPT * NW
E_PAD = NCHUNK * CH

GRP_SHIFT = 11
GROUP = 1 << GRP_SHIFT
NGRP = 5
PAD_DST = 10200
DEGW = 16
ACC_ROWS = 2176
ZR = ACC_ROWS // NS
WR = GROUP // NS


BN = 1000
NB = N // BN


def _xr_body(x_ref, w_ref, o_ref):
    o_ref[...] = jnp.dot(x_ref[...], w_ref[0], preferred_element_type=jnp.float32)


_xr_call = pl.pallas_call(
    _xr_body,
    grid=(NB, R),
    in_specs=[
        pl.BlockSpec((BN, D), lambda nb, r: (nb, 0)),
        pl.BlockSpec((1, D, D), lambda nb, r: (r, 0, 0)),
    ],
    out_specs=pl.BlockSpec((BN, D), lambda nb, r: (r * NB + nb, 0)),
    out_shape=jax.ShapeDtypeStruct((R * N, D), jnp.float32),
)


_sc_mesh = plsc.VectorSubcoreMesh(
    core_axis_name="c", subcore_axis_name="s", num_cores=NC, num_subcores=NS
)


def _make_sc_pass(p):
    @functools.partial(
        pl.kernel,
        out_type=jax.ShapeDtypeStruct((NC, GROUP, D), jnp.float32),
        mesh=_sc_mesh,
        scratch_types=[
            pltpu.VMEM((CPT, CH), jnp.int32),
            pltpu.VMEM((CPT, CH), jnp.int32),
            pltpu.VMEM((CH, D), jnp.float32),
            pltpu.VMEM_SHARED((ACC_ROWS, D), jnp.float32),
            pltpu.SemaphoreType.DMA,
        ],
        name=f"sc_agg_pass{p}",
    )
    def _sc_pass(xr, gix2, dstm2, zacc_h,
                 acc_out,
                 gixb, dstmb, rows, acc_sh, sem0):
        c = lax.axis_index("c")
        s_ = lax.axis_index("s")
        wid = c * NS + s_
        c0 = pl.multiple_of(wid * CPT, 8)

        pltpu.sync_copy(gix2.at[pl.ds(c0, CPT)], gixb)
        pltpu.sync_copy(dstm2.at[pl.ds(c0, CPT)], dstmb)

        z0 = pl.multiple_of(s_ * ZR, 8)
        pltpu.sync_copy(zacc_h, acc_sh.at[pl.ds(z0, ZR)])
        plsc.subcore_barrier()

        @pl.loop(0, CPT)
        def _chunk(j):
            pltpu.async_copy(xr.at[gixb.at[j]], rows, sem0).wait()
            pltpu.sync_copy(rows, acc_sh.at[dstmb.at[j]], add=True)

        plsc.subcore_barrier()

        w0 = pl.multiple_of(s_ * WR, 8)
        pltpu.sync_copy(acc_sh.at[pl.ds(w0, WR)], acc_out.at[c, pl.ds(w0, WR)])

    return _sc_pass


_sc_passes = [_make_sc_pass(p) for p in range(NGRP)]


def _fin_body(acc_ref, dg_ref, x_ref, w_ref, o_ref):
    a = acc_ref[0] + acc_ref[1]
    h = jnp.dot(x_ref[...], w_ref[...], preferred_element_type=jnp.float32)
    o_ref[...] = jnp.maximum(a / dg_ref[...], 0.0) * 0 + jnp.maximum(a / dg_ref[...] + h, 0.0)


def _fin_body2(acc_ref, dg_ref, x_ref, w_ref, o_ref):
    a = acc_ref[0] + acc_ref[1]
    h = jnp.dot(x_ref[...], w_ref[...], preferred_element_type=jnp.float32)
    o_ref[...] = jnp.maximum(a / dg_ref[...] + h, 0.0)


_fin_call = pl.pallas_call(
    _fin_body2,
    grid=(NB,),
    in_specs=[
        pl.BlockSpec((2, BN, D), lambda nb: (0, nb, 0)),
        pl.BlockSpec((BN, 1), lambda nb: (nb, 0)),
        pl.BlockSpec((BN, D), lambda nb: (nb, 0)),
        pl.BlockSpec((D, D), lambda nb: (0, 0)),
    ],
    out_specs=pl.BlockSpec((BN, D), lambda nb: (nb, 0)),
    out_shape=jax.ShapeDtypeStruct((N, D), jnp.float32),
)


def kernel(x, edge_index, edge_type, W_rel, W_self):
    src = edge_index[0]
    dst = edge_index[1]
    pad = E_PAD - E
    src_p = jnp.concatenate([src, jnp.zeros((pad,), jnp.int32)])
    et_p = jnp.concatenate([edge_type, jnp.zeros((pad,), jnp.int32)])
    dst_p = jnp.concatenate([dst, jnp.full((pad,), PAD_DST, jnp.int32)])

    gix2 = (et_p * N + src_p).reshape(NCHUNK, CH)
    dstm2 = []
    for p in range(NGRP):
        inrng = (dst_p >> GRP_SHIFT) == p
        dstm2.append(jnp.where(inrng, dst_p - p * GROUP, GROUP)
                     .astype(jnp.int32).reshape(NCHUNK, CH))

    zacc_h = jnp.zeros((ZR, D), jnp.float32)

    dg = jnp.maximum(
        jax.ops.segment_sum(jnp.ones((E,), jnp.float32), dst, num_segments=N),
        1.0).reshape(N, 1)

    xr = _xr_call(x, W_rel)
    accs = []
    tok = jnp.zeros((), jnp.float32)
    for p in range(NGRP):
        a = _sc_passes[p](xr, gix2, dstm2[p] + (tok * 0).astype(jnp.int32),
                          zacc_h)
        tok = a[0, 0, 0]
        accs.append(a)
    acc = jnp.concatenate(accs, axis=1)[:, :N]
    return _fin_call(acc, dg, x, W_self)


# speedup vs baseline: 2.1178x; 1.0000x over previous
import functools

import jax
import jax.numpy as jnp
from jax import lax
from jax.experimental import pallas as pl
from jax.experimental.pallas import tpu as pltpu
from jax.experimental.pallas import tpu_sc as plsc

N = 10000
E = 320000
D = 128
R = 8

NC = 2
NS = 16
NW = NC * NS

CH = 128
CPT = 8 * -(-E // (CH * NW * 8))
NCHUNK = CPT * NW
E_PAD = NCHUNK * CH

GRP_SHIFT = 11
GROUP = 1 << GRP_SHIFT
NGRP = 5
PAD_DST = 10200
DEGW = 16
ACC_ROWS = 2176
ZR = ACC_ROWS // NS
WR = GROUP // NS


BN = 1000
NB = N // BN


def _xr_body(x_ref, w_ref, o_ref):
    o_ref[...] = jnp.dot(x_ref[...], w_ref[0], preferred_element_type=jnp.float32)


_xr_call = pl.pallas_call(
    _xr_body,
    grid=(NB, R),
    in_specs=[
        pl.BlockSpec((BN, D), lambda nb, r: (nb, 0)),
        pl.BlockSpec((1, D, D), lambda nb, r: (r, 0, 0)),
    ],
    out_specs=pl.BlockSpec((BN, D), lambda nb, r: (r * NB + nb, 0)),
    out_shape=jax.ShapeDtypeStruct((R * N, D), jnp.float32),
)


_sc_mesh = plsc.VectorSubcoreMesh(
    core_axis_name="c", subcore_axis_name="s", num_cores=NC, num_subcores=NS
)


def _make_sc_pass(p):
    @functools.partial(
        pl.kernel,
        out_type=jax.ShapeDtypeStruct((NC, GROUP, D), jnp.float32),
        mesh=_sc_mesh,
        scratch_types=[
            pltpu.VMEM((CPT, CH), jnp.int32),
            pltpu.VMEM((CPT, CH), jnp.int32),
            pltpu.VMEM((CH, D), jnp.float32),
            pltpu.VMEM_SHARED((ACC_ROWS, D), jnp.float32),
            pltpu.SemaphoreType.DMA,
        ],
        name=f"sc_agg_pass{p}",
    )
    def _sc_pass(xr, gix2, dstm2, zacc_h,
                 acc_out,
                 gixb, dstmb, rows, acc_sh, sem0):
        c = lax.axis_index("c")
        s_ = lax.axis_index("s")
        wid = c * NS + s_
        c0 = pl.multiple_of(wid * CPT, 8)

        pltpu.sync_copy(gix2.at[pl.ds(c0, CPT)], gixb)
        pltpu.sync_copy(dstm2.at[pl.ds(c0, CPT)], dstmb)

        z0 = pl.multiple_of(s_ * ZR, 8)
        pltpu.sync_copy(zacc_h, acc_sh.at[pl.ds(z0, ZR)])
        plsc.subcore_barrier()

        @pl.loop(0, CPT)
        def _chunk(j):
            pltpu.async_copy(xr.at[gixb.at[j]], rows, sem0).wait()
            pltpu.sync_copy(rows, acc_sh.at[dstmb.at[j]], add=True)

        plsc.subcore_barrier()

        w0 = pl.multiple_of(s_ * WR, 8)
        pltpu.sync_copy(acc_sh.at[pl.ds(w0, WR)], acc_out.at[c, pl.ds(w0, WR)])

    return _sc_pass


_sc_passes = [_make_sc_pass(p) for p in range(NGRP)]


def _fin_body(acc_ref, dg_ref, x_ref, w_ref, o_ref):
    a = acc_ref[0] + acc_ref[1]
    h = jnp.dot(x_ref[...], w_ref[...], preferred_element_type=jnp.float32)
    o_ref[...] = jnp.maximum(a / dg_ref[...] + h, 0.0)


_fin_call = pl.pallas_call(
    _fin_body,
    grid=(NB,),
    in_specs=[
        pl.BlockSpec((2, BN, D), lambda nb: (0, nb, 0)),
        pl.BlockSpec((BN, 1), lambda nb: (nb, 0)),
        pl.BlockSpec((BN, D), lambda nb: (nb, 0)),
        pl.BlockSpec((D, D), lambda nb: (0, 0)),
    ],
    out_specs=pl.BlockSpec((BN, D), lambda nb: (nb, 0)),
    out_shape=jax.ShapeDtypeStruct((N, D), jnp.float32),
)


def kernel(x, edge_index, edge_type, W_rel, W_self):
    src = edge_index[0]
    dst = edge_index[1]
    pad = E_PAD - E
    src_p = jnp.concatenate([src, jnp.zeros((pad,), jnp.int32)])
    et_p = jnp.concatenate([edge_type, jnp.zeros((pad,), jnp.int32)])
    dst_p = jnp.concatenate([dst, jnp.full((pad,), PAD_DST, jnp.int32)])

    gix2 = (et_p * N + src_p).reshape(NCHUNK, CH)
    dstm2 = []
    for p in range(NGRP):
        inrng = (dst_p >> GRP_SHIFT) == p
        dstm2.append(jnp.where(inrng, dst_p - p * GROUP, GROUP)
                     .astype(jnp.int32).reshape(NCHUNK, CH))

    zacc_h = jnp.zeros((ZR, D), jnp.float32)

    dg = jnp.maximum(
        jax.ops.segment_sum(jnp.ones((E,), jnp.float32), dst, num_segments=N),
        1.0).reshape(N, 1)

    xr = _xr_call(x, W_rel)
    accs = []
    tok = jnp.zeros((), jnp.float32)
    for p in range(NGRP):
        a = _sc_passes[p](xr, gix2, dstm2[p] + (tok * 0).astype(jnp.int32),
                          zacc_h)
        tok = a[0, 0, 0]
        accs.append(a)
    acc = jnp.concatenate(accs, axis=1)[:, :N]
    return _fin_call(acc, dg, x, W_self)
